# Initial kernel scaffold; baseline (speedup 1.0000x reference)
import jax
import jax.numpy as jnp
from jax.experimental import pallas as pl


def kernel(position_sequence, particle_types, edge_index, params):
    raise NotImplementedError("write your pallas kernel here")


# speedup vs baseline: 2.9619x; 2.9619x over previous
import functools

import jax
import jax.numpy as jnp
from jax import lax
from jax.experimental import pallas as pl
from jax.experimental.pallas import tpu as pltpu
from jax.experimental.pallas import tpu_sc as plsc

N = 10000
E = 320000
H = 64
RADIUS = 0.015
INV_R = 1.0 / RADIUS

NC = 2
NS = 16
NW = NC * NS
CHI = 128
NCHUNK = E // CHI
ITER_T = -(-NCHUNK // NS)
ITER_W = -(-NCHUNK // NW)
SHARE = 624
TAIL0 = NS * SHARE
TAIL = N - TAIL0


def _copy_shared(src, dst, sid):
    sl = pl.ds(sid * SHARE, SHARE)
    pltpu.sync_copy(src.at[sl], dst.at[sl])

    @pl.when(sid == 0)
    def _tail():
        tl = pl.ds(TAIL0, TAIL)
        pltpu.sync_copy(src.at[tl], dst.at[tl])

def _relu(x):
    return jnp.maximum(x, 0.0)


def _dot(x, w):
    return jnp.dot(x, w, preferred_element_type=jnp.float32)


def _row_gather_loop(idx_hbm, out_hbm, tbl_sh, idx_v, rows_v, sem, sid):
    def chunk(i, carry):
        c = i * NS + sid

        @pl.when(c < NCHUNK)
        def _go():
            base = c * CHI
            pltpu.sync_copy(idx_hbm.at[pl.ds(base, CHI)], idx_v)
            pltpu.async_copy(tbl_sh.at[idx_v], rows_v, sem).wait()
            pltpu.sync_copy(rows_v, out_hbm.at[pl.ds(base, CHI)])

        return carry

    lax.fori_loop(0, ITER_T, chunk, 0)


def _gather1_body(pt, qt, px, py, snd, rcv, gs, gr, rx_o, ry_o, d2_o,
                  idx_v, idx_s, idx_r, rows_v, px_v, py_v, fx, fy, fd,
                  tbl_sh, sem):
    cid = lax.axis_index("c")
    sid = lax.axis_index("s")

    @pl.when(cid == 0)
    def _stage_p():
        _copy_shared(pt, tbl_sh, sid)

    @pl.when(cid == 1)
    def _stage_q():
        _copy_shared(qt, tbl_sh, sid)

    pltpu.sync_copy(px, px_v)
    pltpu.sync_copy(py, py_v)
    plsc.subcore_barrier()

    @pl.when(cid == 0)
    def _go_s():
        _row_gather_loop(snd, gs, tbl_sh, idx_v, rows_v, sem, sid)

    @pl.when(cid == 1)
    def _go_r():
        _row_gather_loop(rcv, gr, tbl_sh, idx_v, rows_v, sem, sid)

    wid = cid * NS + sid

    def chunk(i, carry):
        c = i * NW + wid

        @pl.when(c < NCHUNK)
        def _go():
            base = c * CHI
            pltpu.sync_copy(snd.at[pl.ds(base, CHI)], idx_s)
            pltpu.sync_copy(rcv.at[pl.ds(base, CHI)], idx_r)

            def vec(k, carry2):
                s = pl.ds(k * 16, 16)
                ivs = idx_s[s]
                ivr = idx_r[s]
                ax = (plsc.load_gather(px_v, [ivs])
                      - plsc.load_gather(px_v, [ivr])) * INV_R
                ay = (plsc.load_gather(py_v, [ivs])
                      - plsc.load_gather(py_v, [ivr])) * INV_R
                fx[s] = ax
                fy[s] = ay
                fd[s] = ax * ax + ay * ay
                return carry2

            lax.fori_loop(0, CHI // 16, vec, 0)
            pltpu.sync_copy(fx, rx_o.at[pl.ds(base, CHI)])
            pltpu.sync_copy(fy, ry_o.at[pl.ds(base, CHI)])
            pltpu.sync_copy(fd, d2_o.at[pl.ds(base, CHI)])

        return carry

    lax.fori_loop(0, ITER_W, chunk, 0)


def _gather2_body(pt, qt, snd, rcv, gs, gr, idx_v, rows_v, tbl_sh, sem):
    cid = lax.axis_index("c")
    sid = lax.axis_index("s")

    @pl.when(cid == 0)
    def _stage_p():
        _copy_shared(pt, tbl_sh, sid)

    @pl.when(cid == 1)
    def _stage_q():
        _copy_shared(qt, tbl_sh, sid)

    plsc.subcore_barrier()

    @pl.when(cid == 0)
    def _go_s():
        _row_gather_loop(snd, gs, tbl_sh, idx_v, rows_v, sem, sid)

    @pl.when(cid == 1)
    def _go_r():
        _row_gather_loop(rcv, gr, tbl_sh, idx_v, rows_v, sem, sid)


def _scatter_body(edge, rcv, zeros, out, idx_v, rows_v, acc):
    cid = lax.axis_index("c")
    sid = lax.axis_index("s")
    _copy_shared(zeros, acc, sid)
    plsc.subcore_barrier()

    wid = cid * NS + sid

    def chunk(i, carry):
        c = i * NW + wid

        @pl.when(c < NCHUNK)
        def _go():
            base = c * CHI
            pltpu.sync_copy(rcv.at[pl.ds(base, CHI)], idx_v)
            pltpu.sync_copy(edge.at[pl.ds(base, CHI)], rows_v)
            pltpu.sync_copy(rows_v, acc.at[idx_v], add=True)

        return carry

    lax.fori_loop(0, ITER_W, chunk, 0)
    plsc.subcore_barrier()
    _copy_shared(acc, out.at[cid], sid)


_f32 = jnp.float32


@functools.lru_cache(maxsize=None)
def _sc_calls():
    mesh = plsc.VectorSubcoreMesh(core_axis_name="c", subcore_axis_name="s")
    gather1 = pl.kernel(
        _gather1_body, mesh=mesh,
        out_type=[jax.ShapeDtypeStruct((E, H), _f32),
                  jax.ShapeDtypeStruct((E, H), _f32),
                  jax.ShapeDtypeStruct((E,), _f32),
                  jax.ShapeDtypeStruct((E,), _f32),
                  jax.ShapeDtypeStruct((E,), _f32)],
        scratch_types=[pltpu.VMEM((CHI,), jnp.int32),
                       pltpu.VMEM((CHI,), jnp.int32),
                       pltpu.VMEM((CHI,), jnp.int32),
                       pltpu.VMEM((CHI, H), _f32),
                       pltpu.VMEM((N,), _f32),
                       pltpu.VMEM((N,), _f32),
                       pltpu.VMEM((CHI,), _f32),
                       pltpu.VMEM((CHI,), _f32),
                       pltpu.VMEM((CHI,), _f32),
                       pltpu.VMEM_SHARED((N, H), _f32),
                       pltpu.SemaphoreType.DMA],
        compiler_params=pltpu.CompilerParams(needs_layout_passes=False),
    )
    gather2 = pl.kernel(
        _gather2_body, mesh=mesh,
        out_type=[jax.ShapeDtypeStruct((E, H), _f32),
                  jax.ShapeDtypeStruct((E, H), _f32)],
        scratch_types=[pltpu.VMEM((CHI,), jnp.int32),
                       pltpu.VMEM((CHI, H), _f32),
                       pltpu.VMEM_SHARED((N, H), _f32),
                       pltpu.SemaphoreType.DMA],
    )
    scatter = pl.kernel(
        _scatter_body, mesh=mesh,
        out_type=[jax.ShapeDtypeStruct((NC, N, H), _f32)],
        scratch_types=[pltpu.VMEM((CHI,), jnp.int32),
                       pltpu.VMEM((CHI, H), _f32),
                       pltpu.VMEM_SHARED((N, H), _f32)],
    )
    return gather1, gather2, scatter


BN = 2000
BE = 4000


def _node_enc_kern(pos12, types, emb, w1, b1, w2, b2, w3, b3, ws, wr,
                   node_o, p_o, q_o):
    p = pos12[...]
    vel = p[:, 2:12] - p[:, 0:10]
    cur = p[:, 10:12]
    db = jnp.concatenate([cur, 1.0 - cur], axis=1) * INV_R
    db = jnp.minimum(db, 1.0)
    t = types[...]
    one = (t == lax.broadcasted_iota(jnp.int32, (BN, 16), 1)).astype(_f32)
    te = _dot(one, emb[...])
    nf = jnp.concatenate([vel, db, te, jnp.zeros((BN, 2), _f32)], axis=1)
    h = _relu(_dot(nf, w1[...]) + b1[...])
    h = _relu(_dot(h, w2[...]) + b2[...])
    node = _dot(h, w3[...]) + b3[...]
    node_o[...] = node
    p_o[...] = _dot(node, ws[...])
    q_o[...] = _dot(node, wr[...])


def _edge1_kern(gs, gr, rx, ry, d2, ew1, eb1, ew2, eb2, ew3, eb3,
                pw1e, pb1, pw2, pb2, pw3, pb3, out):
    d = jnp.sqrt(d2[...])
    w1 = ew1[...]
    h = (rx[...] * w1[0:1, :] + ry[...] * w1[1:2, :]
         + d * w1[2:3, :] + eb1[...])
    h = _relu(h)
    h = _relu(_dot(h, ew2[...]) + eb2[...])
    e0 = _dot(h, ew3[...]) + eb3[...]
    h = _relu(_dot(e0, pw1e[...]) + gs[...] + gr[...] + pb1[...])
    h = _relu(_dot(h, pw2[...]) + pb2[...])
    out[...] = e0 + _dot(h, pw3[...]) + pb3[...]


def _edge2_kern(edge, gs, gr, pw1e, pb1, pw2, pb2, pw3, pb3, out):
    e = edge[...]
    h = _relu(_dot(e, pw1e[...]) + gs[...] + gr[...] + pb1[...])
    h = _relu(_dot(h, pw2[...]) + pb2[...])
    out[...] = e + _dot(h, pw3[...]) + pb3[...]


def _node_upd_kern(node, agg_a, agg_b, wn, wa, b1, w2, b2, w3, b3, ws, wr,
                   node_o, p_o, q_o):
    nd = node[...]
    agg = agg_a[...] + agg_b[...]
    h = _relu(_dot(nd, wn[...]) + _dot(agg, wa[...]) + b1[...])
    h = _relu(_dot(h, w2[...]) + b2[...])
    n1 = nd + _dot(h, w3[...]) + b3[...]
    node_o[...] = n1
    p_o[...] = _dot(n1, ws[...])
    q_o[...] = _dot(n1, wr[...])


def _final_kern(node, agg_a, agg_b, pos12, wn, wa, b1, w2, b2, w3, b3,
                d1, db1, d2, db2, d3t, db3, out):
    nd = node[...]
    agg = agg_a[...] + agg_b[...]
    h = _relu(_dot(nd, wn[...]) + _dot(agg, wa[...]) + b1[...])
    h = _relu(_dot(h, w2[...]) + b2[...])
    n2 = nd + _dot(h, w3[...]) + b3[...]
    h = _relu(_dot(n2, d1[...]) + db1[...])
    h = _relu(_dot(h, d2[...]) + db2[...])
    d3 = d3t[...]
    acc = jnp.concatenate(
        [jnp.sum(h * d3[0:1, :], axis=1, keepdims=True),
         jnp.sum(h * d3[1:2, :], axis=1, keepdims=True)], axis=1) + db3[...]
    p = pos12[...]
    cur = p[:, 10:12]
    prev = p[:, 8:10]
    out[...] = 2.0 * cur - prev + acc


def _full(shape):
    nd = len(shape)
    return pl.BlockSpec(shape, lambda i, _n=nd: (0,) * _n)


def _blk(bs, width):
    return pl.BlockSpec((bs, width), lambda i: (i, 0))


def _tc_call(kern, grid, in_specs, out_specs, out_shape):
    return pl.pallas_call(kern, grid=grid, in_specs=in_specs,
                          out_specs=out_specs, out_shape=out_shape)


def kernel(position_sequence, particle_types, edge_index, params):
    pos12 = position_sequence.reshape(N, 12)
    types = particle_types.astype(jnp.int32).reshape(N, 1)
    receivers = edge_index[0].astype(jnp.int32)
    senders = edge_index[1].astype(jnp.int32)

    emb16 = jnp.zeros((16, 16), _f32).at[:9].set(params["type_embedding"])
    (nw1, nb1), (nw2, nb2), (nw3, nb3) = params["enc_node"]
    nw1p = jnp.zeros((32, H), _f32).at[:30].set(nw1)
    (ew1, eb1), (ew2, eb2), (ew3, eb3) = params["enc_edge"]
    pe = []
    for s in range(2):
        (w1, b1), (w2, b2), (w3, b3) = params["proc_edge"][s]
        pe.append((w1[:H], w1[H:2 * H], w1[2 * H:], b1.reshape(1, H),
                   w2, b2.reshape(1, H), w3, b3.reshape(1, H)))
    pn = []
    for s in range(2):
        (w1, b1), (w2, b2), (w3, b3) = params["proc_node"][s]
        pn.append((w1[:H], w1[H:], b1.reshape(1, H),
                   w2, b2.reshape(1, H), w3, b3.reshape(1, H)))
    (dw1, dbias1), (dw2, dbias2), (dw3, dbias3) = params["dec"]
    d3t = dw3.T

    r1 = lambda b: b.reshape(1, H)

    gridN = (N // BN,)
    node0, p1t, q1t = _tc_call(
        _node_enc_kern, gridN,
        [_blk(BN, 12), _blk(BN, 1), _full((16, 16)),
         _full((32, H)), _full((1, H)), _full((H, H)), _full((1, H)),
         _full((H, H)), _full((1, H)), _full((H, H)), _full((H, H))],
        [_blk(BN, H), _blk(BN, H), _blk(BN, H)],
        [jax.ShapeDtypeStruct((N, H), _f32)] * 3,
    )(pos12, types, emb16, nw1p, r1(nb1), nw2, r1(nb2), nw3, r1(nb3),
      pe[0][1], pe[0][2])

    _gather1_call, _gather2_call, _scatter_call = _sc_calls()
    posx = pos12[:, 10]
    posy = pos12[:, 11]
    gs1, gr1, relx, rely, d2 = _gather1_call(p1t, q1t, posx, posy,
                                             senders, receivers)

    gridE = (E // BE,)
    w1e, _, _, pb1, pw2, pb2, pw3, pb3 = pe[0]
    edge1 = _tc_call(
        _edge1_kern, gridE,
        [_blk(BE, H), _blk(BE, H), _blk(BE, 1), _blk(BE, 1), _blk(BE, 1),
         _full((3, H)), _full((1, H)), _full((H, H)), _full((1, H)),
         _full((H, H)), _full((1, H)),
         _full((H, H)), _full((1, H)), _full((H, H)), _full((1, H)),
         _full((H, H)), _full((1, H))],
        _blk(BE, H), jax.ShapeDtypeStruct((E, H), _f32),
    )(gs1, gr1, relx.reshape(E, 1), rely.reshape(E, 1), d2.reshape(E, 1),
      ew1, r1(eb1), ew2, r1(eb2), ew3, r1(eb3),
      w1e, pb1, pw2, pb2, pw3, pb3)

    zeros_n = jnp.zeros((N, H), _f32)

    agg1 = _scatter_call(edge1, receivers, zeros_n)[0]

    wn, wa, b1, w2, b2, w3, b3 = pn[0]
    node1, p2t, q2t = _tc_call(
        _node_upd_kern, gridN,
        [_blk(BN, H)] * 3
        + [_full((H, H)), _full((H, H)), _full((1, H)), _full((H, H)),
           _full((1, H)), _full((H, H)), _full((1, H)),
           _full((H, H)), _full((H, H))],
        [_blk(BN, H)] * 3,
        [jax.ShapeDtypeStruct((N, H), _f32)] * 3,
    )(node0, agg1[0], agg1[1], wn, wa, b1, w2, b2, w3, b3,
      pe[1][1], pe[1][2])

    gs2, gr2 = _gather2_call(p2t, q2t, senders, receivers)

    w1e2, _, _, pb12, pw22, pb22, pw32, pb32 = pe[1]
    edge2 = _tc_call(
        _edge2_kern, gridE,
        [_blk(BE, H)] * 3
        + [_full((H, H)), _full((1, H)), _full((H, H)), _full((1, H)),
           _full((H, H)), _full((1, H))],
        _blk(BE, H), jax.ShapeDtypeStruct((E, H), _f32),
    )(edge1, gs2, gr2, w1e2, pb12, pw22, pb22, pw32, pb32)

    agg2 = _scatter_call(edge2, receivers, zeros_n)[0]

    wn2, wa2, b12, w22, b22, w32, b32 = pn[1]
    out = _tc_call(
        _final_kern, gridN,
        [_blk(BN, H)] * 3 + [_blk(BN, 12)]
        + [_full((H, H)), _full((H, H)), _full((1, H)), _full((H, H)),
           _full((1, H)), _full((H, H)), _full((1, H)),
           _full((H, H)), _full((1, H)), _full((H, H)), _full((1, H)),
           _full((2, H)), _full((1, 2))],
        _blk(BN, 2), jax.ShapeDtypeStruct((N, 2), _f32),
    )(node1, agg2[0], agg2[1], pos12, wn2, wa2, b12, w22, b22, w32, b32,
      dw1, r1(dbias1), dw2, r1(dbias2), d3t, dbias3.reshape(1, 2))

    return out
